# baseline (speedup 1.0000x reference)
import jax
import jax.numpy as jnp
from jax.experimental import pallas as pl
from jax.experimental.pallas import tpu as pltpu

_TB = 4096
_HB = _TB // 2


def _half(l):
    iota = jax.lax.broadcasted_iota(jnp.int32, l.shape, 1).astype(jnp.float32)
    m1 = jnp.max(l, axis=1, keepdims=True)
    idx1 = jnp.min(jnp.where(l == m1, iota, 64.0), axis=1, keepdims=True)
    hit1 = iota == idx1
    l2 = jnp.where(hit1, -jnp.inf, l)
    m2 = jnp.max(l2, axis=1, keepdims=True)
    idx2 = jnp.min(jnp.where(l2 == m2, iota, 64.0), axis=1, keepdims=True)
    e = jnp.exp(l - m1)
    sinv = 1.0 / jnp.sum(e, axis=1, keepdims=True)
    out = jnp.where(hit1 | (iota == idx2), e * sinv, 0.0)
    idx = jnp.concatenate([idx1, idx2], axis=1).astype(jnp.int32)
    return out, idx


def _router_block(x1_ref, x2_ref, w_ref, out_ref, idx_ref):
    w = w_ref[...]
    dn = (((1,), (1,)), ((), ()))
    l1 = jax.lax.dot_general(x1_ref[0], w, dn, preferred_element_type=jnp.float32)
    o1, i1 = _half(l1)
    out_ref[0, :_HB] = o1
    idx_ref[0, :_HB] = i1
    l2 = jax.lax.dot_general(x2_ref[0], w, dn, preferred_element_type=jnp.float32)
    o2, i2 = _half(l2)
    out_ref[0, _HB:] = o2
    idx_ref[0, _HB:] = i2


@jax.jit
def kernel(x, W):
    B, T, C = x.shape
    E = W.shape[0]
    grid = (B, T // _TB)
    out, idx = pl.pallas_call(
        _router_block,
        grid=grid,
        in_specs=[
            pl.BlockSpec((1, _HB, C), lambda b, i: (b, 2 * i, 0)),
            pl.BlockSpec((1, _HB, C), lambda b, i: (b, 2 * i + 1, 0)),
            pl.BlockSpec((E, C), lambda b, i: (0, 0)),
        ],
        out_specs=[
            pl.BlockSpec((1, _TB, E), lambda b, i: (b, i, 0)),
            pl.BlockSpec((1, _TB, 2), lambda b, i: (b, i, 0)),
        ],
        out_shape=[
            jax.ShapeDtypeStruct((B, T, E), jnp.float32),
            jax.ShapeDtypeStruct((B, T, 2), jnp.int32),
        ],
        compiler_params=pltpu.CompilerParams(
            dimension_semantics=("arbitrary", "arbitrary"),
        ),
    )(x, x, W)
    return out, idx


# speedup vs baseline: 1.0246x; 1.0246x over previous
import jax
import jax.numpy as jnp
from jax.experimental import pallas as pl
from jax.experimental.pallas import tpu as pltpu

_TB = 4096


def _router_block(x_ref, w_ref, out_ref, idx_ref):
    l = jax.lax.dot_general(
        x_ref[0], w_ref[...], (((1,), (1,)), ((), ())),
        preferred_element_type=jnp.float32,
    )
    iota = jax.lax.broadcasted_iota(jnp.int32, l.shape, 1).astype(jnp.float32)
    m1 = jnp.max(l, axis=1, keepdims=True)
    idx1 = jnp.min(jnp.where(l == m1, iota, 64.0), axis=1, keepdims=True)
    hit1 = iota == idx1
    l2 = jnp.where(hit1, -jnp.inf, l)
    m2 = jnp.max(l2, axis=1, keepdims=True)
    idx2 = jnp.min(jnp.where(l2 == m2, iota, 64.0), axis=1, keepdims=True)
    e = jnp.exp(l - m1)
    sinv = 1.0 / jnp.sum(e, axis=1, keepdims=True)
    out_ref[0] = jnp.where(hit1 | (iota == idx2), e * sinv, 0.0)
    idx_ref[0] = jnp.concatenate([idx1, idx2], axis=1).astype(jnp.int32)


@jax.jit
def kernel(x, W):
    B, T, C = x.shape
    E = W.shape[0]
    grid = (B, T // _TB)
    out, idx = pl.pallas_call(
        _router_block,
        grid=grid,
        in_specs=[
            pl.BlockSpec((1, _TB, C), lambda b, i: (b, i, 0)),
            pl.BlockSpec((E, C), lambda b, i: (0, 0)),
        ],
        out_specs=[
            pl.BlockSpec((1, _TB, E), lambda b, i: (b, i, 0)),
            pl.BlockSpec((1, _TB, 2), lambda b, i: (b, i, 0)),
        ],
        out_shape=[
            jax.ShapeDtypeStruct((B, T, E), jnp.float32),
            jax.ShapeDtypeStruct((B, T, 2), jnp.int32),
        ],
        compiler_params=pltpu.CompilerParams(
            dimension_semantics=("arbitrary", "arbitrary"),
        ),
    )(x, W)
    return out, idx
